# baseline (speedup 1.0000x reference)
import functools

import jax
import jax.numpy as jnp
from jax import lax
from jax.experimental import pallas as pl
from jax.experimental.pallas import tpu as pltpu
from jax.experimental.pallas import tpu_sc as plsc

N = 10000
E = 160000
IN_CH = 256
HID_CH = 256
OUT_CH = 128
NUM_GRAPHS = 64

NS = 16
G = 128
NCHUNK = 80
HALF = NCHUNK // 2
E_PAD = NS * NCHUNK * G
N_PAD = 10240
RPT = N_PAD // NS
CB = 128

_mesh = plsc.VectorSubcoreMesh(
    core_axis_name="c", subcore_axis_name="s", num_cores=1, num_subcores=NS
)


@functools.partial(
    pl.kernel,
    out_type=jax.ShapeDtypeStruct((N_PAD,), jnp.float32),
    mesh=_mesh,
    scratch_types=[
        pltpu.VMEM((NCHUNK, G), jnp.int32),
        pltpu.VMEM((G,), jnp.float32),
        pltpu.VMEM((RPT,), jnp.float32),
        pltpu.VMEM_SHARED((N_PAD,), jnp.float32),
    ],
)
def _sc_degree(dst5, out, dst_v, ones_v, zbuf, deg_sh):
    sid = lax.axis_index("s")
    pltpu.sync_copy(dst5.at[sid], dst_v)

    def fill(i, carry):
        ones_v[pl.ds(i * 16, 16)] = jnp.ones((16,), jnp.float32)
        return carry

    lax.fori_loop(0, G // 16, fill, 0)

    def zrow(i, carry):
        zbuf[pl.ds(i * 16, 16)] = jnp.zeros((16,), jnp.float32)
        return carry

    lax.fori_loop(0, RPT // 16, zrow, 0)
    base = sid * RPT
    pltpu.sync_copy(zbuf, deg_sh.at[pl.ds(base, RPT)])
    plsc.subcore_barrier()

    def body(j, carry):
        pltpu.sync_copy(ones_v, deg_sh.at[dst_v.at[j]], add=True)
        return carry

    lax.fori_loop(0, NCHUNK, body, 0)
    plsc.subcore_barrier()
    pltpu.sync_copy(deg_sh.at[pl.ds(base, RPT)], out.at[pl.ds(base, RPT)])


def _agg_one(y, out_ref, src5, dst5, src_v, dst_v, buf0, buf1, acc_sh,
             sem0, sem1, sid):
    def zrow(i, carry):
        for c in range(CB // 16):
            buf0[i, pl.ds(c * 16, 16)] = jnp.zeros((16,), jnp.float32)
        return carry

    lax.fori_loop(0, G, zrow, 0)
    base = sid * RPT
    for k in range(RPT // G):
        pltpu.sync_copy(buf0, acc_sh.at[pl.ds(base + k * G, G)])
    plsc.subcore_barrier()

    def start_g(j, buf, sem):
        pltpu.async_copy(y.at[src_v.at[j]], buf, sem)

    def wait_g(j, buf, sem):
        pltpu.make_async_copy(y.at[src_v.at[j]], buf, sem).wait()

    for h in range(2):
        pltpu.sync_copy(src5.at[sid, pl.ds(h * HALF, HALF)], src_v)
        pltpu.sync_copy(dst5.at[sid, pl.ds(h * HALF, HALF)], dst_v)
        start_g(0, buf0, sem0)

        def body(t, carry):
            j0 = t * 2
            j1 = j0 + 1
            wait_g(j0, buf0, sem0)
            start_g(j1, buf1, sem1)
            pltpu.sync_copy(buf0, acc_sh.at[dst_v.at[j0]], add=True)
            wait_g(j1, buf1, sem1)

            @pl.when(t < HALF // 2 - 1)
            def _():
                start_g(j1 + 1, buf0, sem0)

            pltpu.sync_copy(buf1, acc_sh.at[dst_v.at[j1]], add=True)
            return carry

        lax.fori_loop(0, HALF // 2, body, 0)
    plsc.subcore_barrier()
    pltpu.sync_copy(acc_sh.at[pl.ds(base, RPT)], out_ref.at[pl.ds(base, RPT)])


_SPMM_SCRATCH = [
    pltpu.VMEM((HALF, G), jnp.int32),
    pltpu.VMEM((HALF, G), jnp.int32),
    pltpu.VMEM((G, CB), jnp.float32),
    pltpu.VMEM((G, CB), jnp.float32),
    pltpu.VMEM_SHARED((N_PAD, CB), jnp.float32),
    pltpu.SemaphoreType.DMA,
    pltpu.SemaphoreType.DMA,
]


@functools.partial(
    pl.kernel,
    out_type=jax.ShapeDtypeStruct((N_PAD, CB), jnp.float32),
    mesh=_mesh,
    scratch_types=_SPMM_SCRATCH,
)
def _sc_spmm(y, src5, dst5, out, src_v, dst_v, buf0, buf1, acc_sh, sem0, sem1):
    sid = lax.axis_index("s")
    _agg_one(y, out, src5, dst5, src_v, dst_v, buf0, buf1, acc_sh,
             sem0, sem1, sid)


@functools.partial(
    pl.kernel,
    out_type=jax.ShapeDtypeStruct((2, N_PAD, CB), jnp.float32),
    mesh=_mesh,
    scratch_types=_SPMM_SCRATCH,
)
def _sc_spmm2(ya, yb, src5, dst5, out,
              src_v, dst_v, buf0, buf1, acc_sh, sem0, sem1):
    sid = lax.axis_index("s")
    _agg_one(ya, out.at[0], src5, dst5, src_v, dst_v, buf0, buf1, acc_sh,
             sem0, sem1, sid)
    _agg_one(yb, out.at[1], src5, dst5, src_v, dst_v, buf0, buf1, acc_sh,
             sem0, sem1, sid)


def _tc_prep(degp_ref, x_ref, w1_ref, d_ref, y1a_ref, y1b_ref):
    deg_row = degp_ref[...] + 1.0
    d_row = lax.rsqrt(deg_row)
    d_col = jnp.transpose(jnp.broadcast_to(d_row, (8, N_PAD)))
    d = d_col[:N, 0:1]
    d_ref[...] = d
    h = jnp.dot(x_ref[...], w1_ref[...], preferred_element_type=jnp.float32)
    y = h * d
    y1a_ref[...] = y[:, :CB]
    y1b_ref[...] = y[:, CB:]


def _tc_mid(pa_ref, pb_ref, ya_ref, yb_ref, d_ref, b1_ref, w2_ref, y2_ref):
    d = d_ref[...]
    ha = pa_ref[:N, :] + ya_ref[...]
    hb = pb_ref[:N, :] + yb_ref[...]
    h = jnp.concatenate([ha, hb], axis=1) * d + b1_ref[...]
    h = jnp.maximum(h, 0.0)
    h2 = jnp.dot(h, w2_ref[...], preferred_element_type=jnp.float32)
    y2_ref[...] = h2 * d


def _tc_out(p2_ref, y2_ref, d_ref, b2_ref, batch_ref, pooled_ref):
    h = (p2_ref[:N, :] + y2_ref[...]) * d_ref[...]
    h = h + b2_ref[...]
    gids = lax.broadcasted_iota(jnp.int32, (NUM_GRAPHS, N), 0)
    onehot = (batch_ref[...] == gids).astype(jnp.float32)
    pooled_ref[...] = jnp.dot(onehot, h, preferred_element_type=jnp.float32)


def kernel(x, edge_index, batch, W1, b1, W2, b2):
    src = edge_index[0].astype(jnp.int32)
    dst = edge_index[1].astype(jnp.int32)
    src_f = jnp.concatenate(
        [src, jnp.zeros((E_PAD - E,), jnp.int32)]).reshape(NS, NCHUNK, G)
    dst_f = jnp.concatenate(
        [dst, jnp.full((E_PAD - E,), N, jnp.int32)]).reshape(NS, NCHUNK, G)
    degp = _sc_degree(dst_f).reshape(1, N_PAD)

    d, y1a, y1b = pl.pallas_call(
        _tc_prep,
        out_shape=[
            jax.ShapeDtypeStruct((N, 1), jnp.float32),
            jax.ShapeDtypeStruct((N, CB), jnp.float32),
            jax.ShapeDtypeStruct((N, CB), jnp.float32),
        ],
    )(degp, x, W1)

    pab = _sc_spmm2(y1a, y1b, src_f, dst_f)

    y2 = pl.pallas_call(
        _tc_mid,
        out_shape=jax.ShapeDtypeStruct((N, OUT_CH), jnp.float32),
    )(pab[0], pab[1], y1a, y1b, d, b1.reshape(1, HID_CH), W2)

    p2 = _sc_spmm(y2, src_f, dst_f)

    pooled = pl.pallas_call(
        _tc_out,
        out_shape=jax.ShapeDtypeStruct((NUM_GRAPHS, OUT_CH), jnp.float32),
    )(p2, y2, d, b2.reshape(1, OUT_CH), batch.astype(jnp.int32).reshape(1, N))
    return pooled


# speedup vs baseline: 1.3927x; 1.3927x over previous
import functools

import jax
import jax.numpy as jnp
from jax import lax
from jax.experimental import pallas as pl
from jax.experimental.pallas import tpu as pltpu
from jax.experimental.pallas import tpu_sc as plsc

N = 10000
E = 160000
IN_CH = 256
HID_CH = 256
OUT_CH = 128
NUM_GRAPHS = 64

NC = 2
NS = 16
NW = NC * NS
G = 128
EPT = 5120
NCHUNK = EPT // G
E_PAD = NW * EPT
N_PAD = 10240
RPT = N_PAD // NS
CB = 128
C0 = 64
C1 = 16
CMAX = max(C0, C1)

_mesh = plsc.VectorSubcoreMesh(
    core_axis_name="c", subcore_axis_name="s", num_cores=NC, num_subcores=NS
)


@functools.partial(
    pl.kernel,
    out_type=jax.ShapeDtypeStruct((NC * N_PAD,), jnp.float32),
    mesh=_mesh,
    scratch_types=[
        pltpu.VMEM((NCHUNK, G), jnp.int32),
        pltpu.VMEM((G,), jnp.float32),
        pltpu.VMEM((RPT,), jnp.float32),
        pltpu.VMEM_SHARED((N_PAD,), jnp.float32),
    ],
)
def _sc_degree(dst3, out, dst_v, ones_v, zbuf, deg_sh):
    cid = lax.axis_index("c")
    sid = lax.axis_index("s")
    wid = sid * NC + cid
    pltpu.sync_copy(dst3.at[wid], dst_v)

    def fill(i, carry):
        ones_v[pl.ds(i * 16, 16)] = jnp.ones((16,), jnp.float32)
        return carry

    lax.fori_loop(0, G // 16, fill, 0)

    def zrow(i, carry):
        zbuf[pl.ds(i * 16, 16)] = jnp.zeros((16,), jnp.float32)
        return carry

    lax.fori_loop(0, RPT // 16, zrow, 0)
    base = sid * RPT
    pltpu.sync_copy(zbuf, deg_sh.at[pl.ds(base, RPT)])
    plsc.subcore_barrier()

    def body(j, carry):
        pltpu.sync_copy(ones_v, deg_sh.at[dst_v.at[j]], add=True)
        return carry

    lax.fori_loop(0, NCHUNK, body, 0)
    plsc.subcore_barrier()
    pltpu.sync_copy(deg_sh.at[pl.ds(base, RPT)],
                    out.at[pl.ds(cid * N_PAD + base, RPT)])


def _agg_one(y, out_c, src_v, dst_v, buf0, buf1, acc_sh, sem0, sem1,
             cid, sid, nhalf):
    def zrow(i, carry):
        for c in range(CB // 16):
            buf0[i, pl.ds(c * 16, 16)] = jnp.zeros((16,), jnp.float32)
        return carry

    lax.fori_loop(0, G, zrow, 0)
    base = sid * RPT
    for k in range(RPT // G):
        pltpu.sync_copy(buf0, acc_sh.at[pl.ds(base + k * G, G)])
    plsc.subcore_barrier()

    def start_g(j, buf, sem):
        pltpu.async_copy(y.at[src_v.at[j]], buf, sem)

    def wait_g(j, buf, sem):
        pltpu.make_async_copy(y.at[src_v.at[j]], buf, sem).wait()

    start_g(0, buf0, sem0)

    def body(t, carry):
        j0 = t * 2
        j1 = j0 + 1
        wait_g(j0, buf0, sem0)
        start_g(j1, buf1, sem1)
        pltpu.sync_copy(buf0, acc_sh.at[dst_v.at[j0]], add=True)
        wait_g(j1, buf1, sem1)

        @pl.when(t < nhalf - 1)
        def _():
            start_g(j1 + 1, buf0, sem0)

        pltpu.sync_copy(buf1, acc_sh.at[dst_v.at[j1]], add=True)
        return carry

    lax.fori_loop(0, nhalf, body, 0)
    plsc.subcore_barrier()
    pltpu.sync_copy(acc_sh.at[pl.ds(base, RPT)], out_c.at[pl.ds(base, RPT)])


_SPMM_SCRATCH = [
    pltpu.VMEM((CMAX, G), jnp.int32),
    pltpu.VMEM((CMAX, G), jnp.int32),
    pltpu.VMEM((G, CB), jnp.float32),
    pltpu.VMEM((G, CB), jnp.float32),
    pltpu.VMEM_SHARED((N_PAD, CB), jnp.float32),
    pltpu.SemaphoreType.DMA,
    pltpu.SemaphoreType.DMA,
]


@functools.partial(
    pl.kernel,
    out_type=jax.ShapeDtypeStruct((NC, N_PAD, CB), jnp.float32),
    mesh=_mesh,
    scratch_types=_SPMM_SCRATCH,
)
def _sc_spmm(y, src4, dst4, out, src_v, dst_v, buf0, buf1, acc_sh, sem0, sem1):
    cid = lax.axis_index("c")
    sid = lax.axis_index("s")
    slab = cid * NS + sid
    pltpu.sync_copy(src4.at[slab], src_v)
    pltpu.sync_copy(dst4.at[slab], dst_v)
    nhalf = jnp.where(cid == 0, C0 // 2, C1 // 2)
    _agg_one(y, out.at[cid], src_v, dst_v, buf0, buf1, acc_sh, sem0, sem1,
             cid, sid, nhalf)


@functools.partial(
    pl.kernel,
    out_type=jax.ShapeDtypeStruct((2, NC, N_PAD, CB), jnp.float32),
    mesh=_mesh,
    scratch_types=_SPMM_SCRATCH,
)
def _sc_spmm2(ya, yb, src4, dst4, out,
              src_v, dst_v, buf0, buf1, acc_sh, sem0, sem1):
    cid = lax.axis_index("c")
    sid = lax.axis_index("s")
    slab = cid * NS + sid
    pltpu.sync_copy(src4.at[slab], src_v)
    pltpu.sync_copy(dst4.at[slab], dst_v)
    nhalf = jnp.where(cid == 0, C0 // 2, C1 // 2)
    _agg_one(ya, out.at[0, cid], src_v, dst_v, buf0, buf1, acc_sh, sem0, sem1,
             cid, sid, nhalf)
    _agg_one(yb, out.at[1, cid], src_v, dst_v, buf0, buf1, acc_sh, sem0, sem1,
             cid, sid, nhalf)


def _tc_prep(degp_ref, x_ref, w1_ref, d_ref, y1a_ref, y1b_ref):
    deg_row = degp_ref[0:1, :] + degp_ref[1:2, :] + 1.0
    d_row = lax.rsqrt(deg_row)
    d_col = jnp.transpose(jnp.broadcast_to(d_row, (8, N_PAD)))
    d = d_col[:N, 0:1]
    d_ref[...] = d
    h = jnp.dot(x_ref[...], w1_ref[...], preferred_element_type=jnp.float32)
    y = h * d
    y1a_ref[...] = y[:, :CB]
    y1b_ref[...] = y[:, CB:]


def _tc_mid(pa_ref, pb_ref, ya_ref, yb_ref, d_ref, b1_ref, w2_ref, y2_ref):
    d = d_ref[...]
    ha = pa_ref[0, :N, :] + pa_ref[1, :N, :] + ya_ref[...]
    hb = pb_ref[0, :N, :] + pb_ref[1, :N, :] + yb_ref[...]
    h = jnp.concatenate([ha, hb], axis=1) * d + b1_ref[...]
    h = jnp.maximum(h, 0.0)
    h2 = jnp.dot(h, w2_ref[...], preferred_element_type=jnp.float32)
    y2_ref[...] = h2 * d


def _tc_out(p2_ref, y2_ref, d_ref, b2_ref, batch_ref, pooled_ref):
    h = (p2_ref[0, :N, :] + p2_ref[1, :N, :] + y2_ref[...]) * d_ref[...]
    h = h + b2_ref[...]
    gids = lax.broadcasted_iota(jnp.int32, (NUM_GRAPHS, N), 0)
    onehot = (batch_ref[...] == gids).astype(jnp.float32)
    pooled_ref[...] = jnp.dot(onehot, h, preferred_element_type=jnp.float32)


def kernel(x, edge_index, batch, W1, b1, W2, b2):
    src = edge_index[0].astype(jnp.int32)
    dst = edge_index[1].astype(jnp.int32)
    src_p = jnp.concatenate(
        [src, jnp.zeros((E_PAD - E,), jnp.int32)]).reshape(NW, NCHUNK, G)
    dst_p = jnp.concatenate(
        [dst, jnp.full((E_PAD - E,), N, jnp.int32)]).reshape(NW, NCHUNK, G)
    degp = _sc_degree(dst_p).reshape(NC, N_PAD)

    d, y1a, y1b = pl.pallas_call(
        _tc_prep,
        out_shape=[
            jax.ShapeDtypeStruct((N, 1), jnp.float32),
            jax.ShapeDtypeStruct((N, CB), jnp.float32),
            jax.ShapeDtypeStruct((N, CB), jnp.float32),
        ],
    )(degp, x, W1)

    def slabify(flat, fill):
        f = flat.reshape(E_PAD // G, G)

        def pad(v, c):
            if c == CMAX:
                return v
            return jnp.concatenate(
                [v, jnp.full((NS, CMAX - c, G), fill, jnp.int32)], axis=1)

        a = pad(f[: NS * C0].reshape(NS, C0, G), C0)
        b = pad(f[NS * C0:].reshape(NS, C1, G), C1)
        return jnp.concatenate([a, b], axis=0)

    src_f = slabify(src_p, 0)
    dst_f = slabify(dst_p, N)
    pab = _sc_spmm2(y1a, y1b, src_f, dst_f)

    y2 = pl.pallas_call(
        _tc_mid,
        out_shape=jax.ShapeDtypeStruct((N, OUT_CH), jnp.float32),
    )(pab[0], pab[1], y1a, y1b, d, b1.reshape(1, HID_CH), W2)

    p2 = _sc_spmm(y2, src_f, dst_f)

    pooled = pl.pallas_call(
        _tc_out,
        out_shape=jax.ShapeDtypeStruct((NUM_GRAPHS, OUT_CH), jnp.float32),
    )(p2, y2, d, b2.reshape(1, OUT_CH), batch.astype(jnp.int32).reshape(1, N))
    return pooled
